# Initial kernel scaffold; baseline (speedup 1.0000x reference)
import jax
import jax.numpy as jnp
from jax.experimental import pallas as pl


def kernel(ins, batch):
    raise NotImplementedError("write your pallas kernel here")


# speedup vs baseline: 4.3847x; 4.3847x over previous
import functools

import jax
import jax.numpy as jnp
from jax import lax
from jax.experimental import pallas as pl
from jax.experimental.pallas import tpu as pltpu
from jax.experimental.pallas import tpu_sc as plsc

N = 320000
D = 128
S = 1024
CHUNK = 128
NCHUNKS = N // CHUNK
NC = 2
NS = 16
CW = 128
CHUNKS_PER_SC = NCHUNKS // NC
BASE_PER_TILE = CHUNKS_PER_SC // NS
EXTRA = CHUNKS_PER_SC - BASE_PER_TILE * NS
SEGS_PER_TILE = S // NS


def _accumulate():
    mesh = plsc.VectorSubcoreMesh(core_axis_name="c", subcore_axis_name="s")

    @functools.partial(
        pl.kernel,
        mesh=mesh,
        out_type=[
            jax.ShapeDtypeStruct((NC, S, D), jnp.float32),
            jax.ShapeDtypeStruct((NC, S, CW), jnp.float32),
        ],
        scratch_types=[
            pltpu.VMEM((CHUNK,), jnp.int32),
            pltpu.VMEM((CHUNK, D), jnp.float32),
            pltpu.VMEM((CHUNK, CW), jnp.float32),
            pltpu.VMEM((SEGS_PER_TILE, CW), jnp.float32),
            pltpu.VMEM_SHARED((S, D), jnp.float32),
            pltpu.VMEM_SHARED((S, CW), jnp.float32),
        ],
    )
    def body(ins_hbm, ids_hbm, sums_out, cnt_out,
             idx_v, rows_v, ones_v, zc_v, acc_sh, cnt_sh):
        c = lax.axis_index("c")
        s = lax.axis_index("s")
        zero16 = jnp.zeros((16,), jnp.float32)
        one16 = jnp.full((16,), 1.0, jnp.float32)

        def z_rows(i, _):
            r = i // (D // 16)
            q = i % (D // 16)
            rows_v[r, pl.ds(q * 16, 16)] = zero16
            return 0
        lax.fori_loop(0, SEGS_PER_TILE * (D // 16), z_rows, 0)

        def z_cnt(i, _):
            r = i // (CW // 16)
            q = i % (CW // 16)
            zc_v[r, pl.ds(q * 16, 16)] = zero16
            return 0
        lax.fori_loop(0, SEGS_PER_TILE * (CW // 16), z_cnt, 0)

        def f_ones(i, _):
            r = i // (CW // 16)
            q = i % (CW // 16)
            ones_v[r, pl.ds(q * 16, 16)] = one16
            return 0
        lax.fori_loop(0, CHUNK * (CW // 16), f_ones, 0)

        seg0 = s * SEGS_PER_TILE
        pltpu.sync_copy(rows_v.at[pl.ds(0, SEGS_PER_TILE)],
                        acc_sh.at[pl.ds(seg0, SEGS_PER_TILE)])
        pltpu.sync_copy(zc_v, cnt_sh.at[pl.ds(seg0, SEGS_PER_TILE)])
        plsc.subcore_barrier()

        start = c * CHUNKS_PER_SC + s * BASE_PER_TILE + jnp.minimum(s, EXTRA)
        count = BASE_PER_TILE + jnp.where(s < EXTRA, 1, 0)

        def chunk_body(j, _):
            g = start + j
            pltpu.sync_copy(ids_hbm.at[g], idx_v)
            pltpu.sync_copy(ins_hbm.at[pl.ds(g * CHUNK, CHUNK)], rows_v)
            pltpu.sync_copy(rows_v, acc_sh.at[idx_v], add=True)
            pltpu.sync_copy(ones_v, cnt_sh.at[idx_v], add=True)
            return 0
        lax.fori_loop(0, count, chunk_body, 0)
        plsc.subcore_barrier()

        pltpu.sync_copy(acc_sh.at[pl.ds(seg0, SEGS_PER_TILE)],
                        rows_v.at[pl.ds(0, SEGS_PER_TILE)])
        pltpu.sync_copy(rows_v.at[pl.ds(0, SEGS_PER_TILE)],
                        sums_out.at[c, pl.ds(seg0, SEGS_PER_TILE)])
        pltpu.sync_copy(cnt_sh.at[pl.ds(seg0, SEGS_PER_TILE)], zc_v)
        pltpu.sync_copy(zc_v, cnt_out.at[c, pl.ds(seg0, SEGS_PER_TILE)])

    return body


def _finalize_body(sp_ref, cp_ref, o_ref):
    sums = sp_ref[0] + sp_ref[1]
    cnts = cp_ref[0] + cp_ref[1]
    o_ref[...] = sums / jnp.maximum(cnts[:, 0:1], 1.0)


def kernel(ins, batch):
    ids = batch.astype(jnp.int32).reshape(NCHUNKS, CHUNK)
    sums_p, cnt_p = _accumulate()(ins, ids)
    return pl.pallas_call(
        _finalize_body,
        out_shape=jax.ShapeDtypeStruct((S, D), jnp.float32),
    )(sums_p, cnt_p)


# speedup vs baseline: 8.4257x; 1.9216x over previous
import functools

import jax
import jax.numpy as jnp
from jax import lax
from jax.experimental import pallas as pl
from jax.experimental.pallas import tpu as pltpu
from jax.experimental.pallas import tpu_sc as plsc

N = 320000
D = 128
S = 1024
CHUNK = 256
SUB = 128
NCHUNKS = N // CHUNK
NC = 2
NS = 16
CHUNKS_PER_SC = NCHUNKS // NC
BASE_PER_TILE = CHUNKS_PER_SC // NS
EXTRA = CHUNKS_PER_SC - BASE_PER_TILE * NS
SEGS_PER_TILE = S // NS


def _accumulate():
    mesh = plsc.VectorSubcoreMesh(core_axis_name="c", subcore_axis_name="s")

    @functools.partial(
        pl.kernel,
        mesh=mesh,
        out_type=[
            jax.ShapeDtypeStruct((NC, S, D), jnp.float32),
            jax.ShapeDtypeStruct((NC, NS, S), jnp.float32),
        ],
        scratch_types=[
            pltpu.VMEM((CHUNK // SUB, SUB), jnp.int32),
            pltpu.VMEM((CHUNK // SUB, SUB), jnp.int32),
            pltpu.VMEM((CHUNK, D), jnp.float32),
            pltpu.VMEM((CHUNK, D), jnp.float32),
            pltpu.VMEM((S + 16,), jnp.float32),
            pltpu.VMEM_SHARED((S, D), jnp.float32),
            pltpu.SemaphoreType.DMA,
            pltpu.SemaphoreType.DMA,
            pltpu.SemaphoreType.DMA,
            pltpu.SemaphoreType.DMA,
        ],
    )
    def body(ins_hbm, ids_hbm, sums_out, cnt_out,
             idx_v0, idx_v1, rows_v0, rows_v1, cnt_v, acc_sh,
             rsem0, rsem1, isem0, isem1):
        c = lax.axis_index("c")
        s = lax.axis_index("s")
        zero16 = jnp.zeros((16,), jnp.float32)
        ilane = lax.iota(jnp.int32, 16)
        inc16 = jnp.where(ilane == 0, 16.0, 0.0).astype(jnp.float32)
        inc1 = jnp.where(ilane == 0, 1.0, 0.0).astype(jnp.float32)
        idx_bufs = (idx_v0, idx_v1)
        rows_bufs = (rows_v0, rows_v1)
        rsems = (rsem0, rsem1)
        isems = (isem0, isem1)

        def z_rows(i, _):
            rows_v0[i // 8, pl.ds((i % 8) * 16, 16)] = zero16
            return 0
        lax.fori_loop(0, SEGS_PER_TILE * (D // 16), z_rows, 0)

        def z_cnt(i, _):
            cnt_v[pl.ds(i * 16, 16)] = zero16
            return 0
        lax.fori_loop(0, (S + 16) // 16, z_cnt, 0)

        seg0 = s * SEGS_PER_TILE
        pltpu.sync_copy(rows_v0.at[pl.ds(0, SEGS_PER_TILE)],
                        acc_sh.at[pl.ds(seg0, SEGS_PER_TILE)])
        plsc.subcore_barrier()

        start = c * CHUNKS_PER_SC + s * BASE_PER_TILE + jnp.minimum(s, EXTRA)
        count = BASE_PER_TILE + jnp.where(s < EXTRA, 1, 0)

        def issue(b, g):
            @pl.when(g < count)
            def _():
                gg = start + g
                pltpu.async_copy(ins_hbm.at[pl.ds(gg * CHUNK, CHUNK)],
                                 rows_bufs[b], rsems[b])
                pltpu.async_copy(ids_hbm.at[gg], idx_bufs[b], isems[b])

        issue(0, 0)
        issue(1, 1)

        def outer(j2, _):
            for b in range(2):
                g = j2 * 2 + b

                @pl.when(g < count)
                def _():
                    pltpu.make_async_copy(ins_hbm.at[pl.ds(0, CHUNK)],
                                          rows_bufs[b], rsems[b]).wait()
                    pltpu.make_async_copy(ids_hbm.at[0],
                                          idx_bufs[b], isems[b]).wait()
                    for q in range(CHUNK // SUB):
                        pltpu.sync_copy(rows_bufs[b].at[pl.ds(q * SUB, SUB)],
                                        acc_sh.at[idx_bufs[b].at[q]],
                                        add=True)
                    for q in range(CHUNK // SUB):
                        def cnt_sub(t, u, _q=q, _b=b):
                            span = idx_bufs[_b][_q, pl.ds(t * 16, 16)]
                            a = span[0]
                            z = span[15]

                            def uniform():
                                w = cnt_v[pl.ds(a, 16)]
                                cnt_v[pl.ds(a, 16)] = w + inc16

                            def mixed():
                                for l in range(16):
                                    v = span[l]
                                    w = cnt_v[pl.ds(v, 16)]
                                    cnt_v[pl.ds(v, 16)] = w + inc1

                            lax.cond(a == z, uniform, mixed)
                            return u
                        lax.fori_loop(0, SUB // 16, cnt_sub, 0)
                    issue(b, g + 2)
            return 0
        lax.fori_loop(0, (count + 1) // 2, outer, 0)
        plsc.subcore_barrier()

        pltpu.sync_copy(acc_sh.at[pl.ds(seg0, SEGS_PER_TILE)],
                        rows_v0.at[pl.ds(0, SEGS_PER_TILE)])
        pltpu.sync_copy(rows_v0.at[pl.ds(0, SEGS_PER_TILE)],
                        sums_out.at[c, pl.ds(seg0, SEGS_PER_TILE)])
        pltpu.sync_copy(cnt_v.at[pl.ds(0, S)], cnt_out.at[c, s])

    return body


def _finalize_body(sp_ref, cp_ref, o_ref):
    sums = sp_ref[0] + sp_ref[1]
    cnts = jnp.sum(cp_ref[...], axis=(0, 1))
    o_ref[...] = sums / jnp.maximum(cnts[:, None], 1.0)


def kernel(ins, batch):
    ids = batch.astype(jnp.int32).reshape(NCHUNKS, CHUNK // SUB, SUB)
    sums_p, cnt_p = _accumulate()(ins, ids)
    return pl.pallas_call(
        _finalize_body,
        out_shape=jax.ShapeDtypeStruct((S, D), jnp.float32),
    )(sums_p, cnt_p)


# speedup vs baseline: 10.9903x; 1.3044x over previous
import functools

import jax
import jax.numpy as jnp
from jax import lax
from jax.experimental import pallas as pl
from jax.experimental.pallas import tpu as pltpu
from jax.experimental.pallas import tpu_sc as plsc

N = 320000
D = 128
S = 1024
CHUNK = 256
SUB = 128
SPAN = 16
NCHUNKS = N // CHUNK
NC = 2
NS = 16
CHUNKS_PER_SC = NCHUNKS // NC
BASE_PER_TILE = CHUNKS_PER_SC // NS
EXTRA = CHUNKS_PER_SC - BASE_PER_TILE * NS
SEGS_PER_TILE = S // NS


def _accumulate():
    mesh = plsc.VectorSubcoreMesh(core_axis_name="c", subcore_axis_name="s")

    @functools.partial(
        pl.kernel,
        mesh=mesh,
        out_type=[
            jax.ShapeDtypeStruct((NC, S, D), jnp.float32),
            jax.ShapeDtypeStruct((NC, NS, S), jnp.float32),
        ],
        scratch_types=[
            pltpu.VMEM((CHUNK // SUB, SUB), jnp.int32),
            pltpu.VMEM((CHUNK // SUB, SUB), jnp.int32),
            pltpu.VMEM((CHUNK, D), jnp.float32),
            pltpu.VMEM((CHUNK, D), jnp.float32),
            pltpu.VMEM((S + 16,), jnp.float32),
            pltpu.VMEM((SPAN, D), jnp.float32),
            pltpu.VMEM((SPAN,), jnp.int32),
            pltpu.VMEM((D,), jnp.float32),
            pltpu.VMEM_SHARED((S, D), jnp.float32),
            pltpu.SemaphoreType.DMA,
            pltpu.SemaphoreType.DMA,
            pltpu.SemaphoreType.DMA,
            pltpu.SemaphoreType.DMA,
        ],
    )
    def body(ins_hbm, ids_hbm, sums_out, cnt_out,
             idx_v0, idx_v1, rows_v0, rows_v1, cnt_v, flush_v, fidx_v,
             run_v, acc_sh, rsem0, rsem1, isem0, isem1):
        c = lax.axis_index("c")
        s = lax.axis_index("s")
        zero16 = jnp.zeros((16,), jnp.float32)
        ilane = lax.iota(jnp.int32, 16)
        inc16 = jnp.where(ilane == 0, 16.0, 0.0).astype(jnp.float32)
        inc1 = jnp.where(ilane == 0, 1.0, 0.0).astype(jnp.float32)
        zeros_acc = (zero16,) * (D // 16)
        idx_bufs = (idx_v0, idx_v1)
        rows_bufs = (rows_v0, rows_v1)
        rsems = (rsem0, rsem1)
        isems = (isem0, isem1)

        def z_rows(i, _):
            rows_v0[i // 8, pl.ds((i % 8) * 16, 16)] = zero16
            return 0
        lax.fori_loop(0, SEGS_PER_TILE * (D // 16), z_rows, 0)

        def z_cnt(i, _):
            cnt_v[pl.ds(i * 16, 16)] = zero16
            return 0
        lax.fori_loop(0, (S + 16) // 16, z_cnt, 0)

        def z_flush(i, _):
            flush_v[i // 8, pl.ds((i % 8) * 16, 16)] = zero16
            return 0
        lax.fori_loop(0, SPAN * (D // 16), z_flush, 0)

        seg0 = s * SEGS_PER_TILE
        pltpu.sync_copy(rows_v0.at[pl.ds(0, SEGS_PER_TILE)],
                        acc_sh.at[pl.ds(seg0, SEGS_PER_TILE)])
        plsc.subcore_barrier()

        start = c * CHUNKS_PER_SC + s * BASE_PER_TILE + jnp.minimum(s, EXTRA)
        count = BASE_PER_TILE + jnp.where(s < EXTRA, 1, 0)

        def issue(b, g):
            @pl.when(g < count)
            def _():
                gg = start + g
                pltpu.async_copy(ins_hbm.at[pl.ds(gg * CHUNK, CHUNK)],
                                 rows_bufs[b], rsems[b])
                pltpu.async_copy(ids_hbm.at[gg], idx_bufs[b], isems[b])

        issue(0, 0)
        issue(1, 1)

        def flush(rid):
            @pl.when(rid >= 0)
            def _():
                for k in range(D // 16):
                    flush_v[0, pl.ds(k * 16, 16)] = run_v[pl.ds(k * 16, 16)]
                fidx_v[...] = jnp.full((SPAN,), 0, jnp.int32) + rid
                pltpu.sync_copy(flush_v, acc_sh.at[fidx_v], add=True)

        def process_block(b, g, rid_in):
            pltpu.make_async_copy(ins_hbm.at[pl.ds(0, CHUNK)],
                                  rows_bufs[b], rsems[b]).wait()
            pltpu.make_async_copy(ids_hbm.at[0],
                                  idx_bufs[b], isems[b]).wait()
            rid_cur = rid_in
            for q in range(CHUNK // SUB):
                def span_body(t, rid, _q=q, _b=b):
                    span = idx_bufs[_b][_q, pl.ds(t * SPAN, 16)]
                    a = span[0]
                    z = span[15]
                    r0 = _q * SUB + t * SPAN

                    def uniform():
                        red = []
                        for k in range(D // 16):
                            v = rows_bufs[_b][r0, pl.ds(k * 16, 16)]
                            for r in range(1, SPAN):
                                v = v + rows_bufs[_b][r0 + r,
                                                      pl.ds(k * 16, 16)]
                            red.append(v)
                        w = cnt_v[pl.ds(a, 16)]
                        cnt_v[pl.ds(a, 16)] = w + inc16

                        def same_fn():
                            for k in range(D // 16):
                                run_v[pl.ds(k * 16, 16)] = (
                                    run_v[pl.ds(k * 16, 16)] + red[k])

                        def diff_fn():
                            flush(rid)
                            for k in range(D // 16):
                                run_v[pl.ds(k * 16, 16)] = red[k]

                        lax.cond(a == rid, same_fn, diff_fn)
                        return a

                    def mixed():
                        flush(rid)
                        fidx_v[...] = span
                        pltpu.sync_copy(rows_bufs[_b].at[pl.ds(r0, SPAN)],
                                        acc_sh.at[fidx_v], add=True)
                        for l in range(SPAN):
                            v = span[l]
                            w = cnt_v[pl.ds(v, 16)]
                            cnt_v[pl.ds(v, 16)] = w + inc1
                        return jnp.int32(-1)

                    return lax.cond(a == z, uniform, mixed)

                rid_cur = lax.fori_loop(0, SUB // SPAN, span_body, rid_cur)
            issue(b, g + 2)
            return rid_cur

        def outer(j2, rid):
            for b in range(2):
                g = j2 * 2 + b
                rid = lax.cond(
                    g < count,
                    functools.partial(process_block, b, g),
                    lambda r: r,
                    rid)
            return rid

        rid = lax.fori_loop(0, (count + 1) // 2, outer, jnp.int32(-1))
        flush(rid)
        plsc.subcore_barrier()

        pltpu.sync_copy(acc_sh.at[pl.ds(seg0, SEGS_PER_TILE)],
                        rows_v0.at[pl.ds(0, SEGS_PER_TILE)])
        pltpu.sync_copy(rows_v0.at[pl.ds(0, SEGS_PER_TILE)],
                        sums_out.at[c, pl.ds(seg0, SEGS_PER_TILE)])
        pltpu.sync_copy(cnt_v.at[pl.ds(0, S)], cnt_out.at[c, s])

    return body


def _finalize_body(sp_ref, cp_ref, o_ref):
    sums = sp_ref[0] + sp_ref[1]
    cnts = jnp.sum(cp_ref[...], axis=(0, 1))
    o_ref[...] = sums / jnp.maximum(cnts[:, None], 1.0)


def kernel(ins, batch):
    ids = batch.astype(jnp.int32).reshape(NCHUNKS, CHUNK // SUB, SUB)
    sums_p, cnt_p = _accumulate()(ins, ids)
    return pl.pallas_call(
        _finalize_body,
        out_shape=jax.ShapeDtypeStruct((S, D), jnp.float32),
    )(sums_p, cnt_p)
